# Initial kernel scaffold; baseline (speedup 1.0000x reference)
import jax
import jax.numpy as jnp
from jax.experimental import pallas as pl


def kernel(boxes, scores):
    raise NotImplementedError("write your pallas kernel here")


# speedup vs baseline: 6.0445x; 6.0445x over previous
import functools

import jax
import jax.numpy as jnp
from jax import lax
from jax.experimental import pallas as pl
from jax.experimental.pallas import tpu as pltpu
from jax.experimental.pallas import tpu_sc as plsc

_N = 5000
_NPAD = 5120
_L = 16
_HALF_IOU = 0.5


def _nms_body(x1h, y1h, x2h, y2h, keep_h,
              x1, y1, x2, y2, kx1, ky1, kx2, ky2, kar, keepv):
    cid = lax.axis_index("c")
    sid = lax.axis_index("s")

    @pl.when((cid == 0) & (sid == 0))
    def _():
        pltpu.sync_copy(x1h, x1)
        pltpu.sync_copy(y1h, y1)
        pltpu.sync_copy(x2h, x2)
        pltpu.sync_copy(y2h, y2)

        lanes = lax.broadcasted_iota(jnp.int32, (_L,), 0)
        lane0 = lanes == 0
        ffalse = lanes < 0

        def init(i, carry):
            sl = pl.ds(i * _L, _L)
            kx1[sl] = jnp.full((_L,), 2e9, jnp.float32)
            ky1[sl] = jnp.full((_L,), 2e9, jnp.float32)
            kx2[sl] = jnp.full((_L,), 2e9, jnp.float32)
            ky2[sl] = jnp.full((_L,), 2e9, jnp.float32)
            kar[sl] = jnp.full((_L,), 1.0, jnp.float32)
            keepv[sl] = jnp.zeros((_L,), jnp.float32)
            return carry

        lax.fori_loop(0, _NPAD // _L, init, 0)

        def cand_body(j, k_count):
            jv = jnp.full((_L,), j, jnp.int32)
            cx1 = plsc.load_gather(x1, [jv])
            cy1 = plsc.load_gather(y1, [jv])
            cx2 = plsc.load_gather(x2, [jv])
            cy2 = plsc.load_gather(y2, [jv])
            car = (cx2 - cx1) * (cy2 - cy1)

            def inner(t, sup):
                sl = pl.ds(t * _L, _L)
                xx1 = jnp.maximum(cx1, kx1[sl])
                yy1 = jnp.maximum(cy1, ky1[sl])
                xx2 = jnp.minimum(cx2, kx2[sl])
                yy2 = jnp.minimum(cy2, ky2[sl])
                w = jnp.maximum(xx2 - xx1, 0.0)
                h = jnp.maximum(yy2 - yy1, 0.0)
                inter = w * h
                union = car + kar[sl] - inter
                return sup | (inter > _HALF_IOU * union)

            nvec = (k_count + _L - 1) // _L
            sup = lax.fori_loop(0, nvec, inner, ffalse)
            suppressed = jnp.any(sup)

            def append():
                kv = jnp.full((_L,), k_count, jnp.int32)
                plsc.store_scatter(kx1, [kv], cx1, mask=lane0)
                plsc.store_scatter(ky1, [kv], cy1, mask=lane0)
                plsc.store_scatter(kx2, [kv], cx2, mask=lane0)
                plsc.store_scatter(ky2, [kv], cy2, mask=lane0)
                plsc.store_scatter(kar, [kv], car, mask=lane0)
                plsc.store_scatter(keepv, [jv], jnp.full((_L,), 1.0, jnp.float32),
                                  mask=lane0)
                return k_count + 1

            return lax.cond(suppressed, lambda: k_count, append)

        lax.fori_loop(0, _N, cand_body, 0)
        pltpu.sync_copy(keepv, keep_h)


@jax.jit
def _nms_keep01(x1s, y1s, x2s, y2s):
    mesh = plsc.VectorSubcoreMesh(core_axis_name="c", subcore_axis_name="s")
    f = pl.kernel(
        _nms_body,
        out_type=jax.ShapeDtypeStruct((_NPAD,), jnp.float32),
        mesh=mesh,
        scratch_types=[pltpu.VMEM((_NPAD,), jnp.float32) for _ in range(4)]
        + [pltpu.VMEM((_NPAD,), jnp.float32) for _ in range(5)]
        + [pltpu.VMEM((_NPAD,), jnp.float32)],
        compiler_params=pltpu.CompilerParams(needs_layout_passes=False),
    )
    return f(x1s, y1s, x2s, y2s)


def kernel(boxes, scores):
    order = jnp.argsort(-scores)
    boxes_sorted = boxes[order]
    scores_sorted = scores[order]
    pad = _NPAD - boxes_sorted.shape[0]
    bp = jnp.pad(boxes_sorted, ((0, pad), (0, 0)))
    keep01 = _nms_keep01(bp[:, 0], bp[:, 1], bp[:, 2], bp[:, 3])[:_N]
    keep = keep01 > 0.5
    kept_scores = scores_sorted * keep01
    return kept_scores, keep, order


# speedup vs baseline: 29.6904x; 4.9120x over previous
import jax
import jax.numpy as jnp
from jax import lax
from jax.experimental import pallas as pl
from jax.experimental.pallas import tpu as pltpu
from jax.experimental.pallas import tpu_sc as plsc

_N = 5000
_NPAD = 5120
_L = 16
_NW = 16
_U = 2
_B = _NW * _L * _U
_NB = _NPAD // _B
_FAR = 2e9


def _splat_gather(ref, idx_scalar):
    iv = jnp.full((_L,), idx_scalar, jnp.int32)
    return plsc.load_gather(ref, [iv])


def _nms_body(x1h, y1h, x2h, y2h, keep_h,
              x1, y1, x2, y2,
              kx1, ky1, kx2, ky2, kar,
              bx1, by1, bx2, by2, bar_, bidx_l, alive_l, keepv,
              stage, knew_l,
              alive_sh, bidx_sh, knew_sh):
    w = lax.axis_index("s")
    lanes = lax.broadcasted_iota(jnp.int32, (_L,), 0)
    lane0 = lanes == 0
    ffalse = lanes < 0
    fone = jnp.full((_L,), 1.0, jnp.float32)

    pltpu.sync_copy(x1h, x1)
    pltpu.sync_copy(y1h, y1)
    pltpu.sync_copy(x2h, x2)
    pltpu.sync_copy(y2h, y2)

    @pl.when(w == 0)
    def _():
        def initk(i, c):
            keepv[pl.ds(i * _L, _L)] = jnp.zeros((_L,), jnp.float32)
            return c
        lax.fori_loop(0, _NPAD // _L, initk, 0)

    def block_body(jb, k_count):
        base = jb * _B
        mybase = base + w * (_L * _U)

        ca_x1 = x1[pl.ds(mybase, _L)]
        ca_y1 = y1[pl.ds(mybase, _L)]
        ca_x2 = x2[pl.ds(mybase, _L)]
        ca_y2 = y2[pl.ds(mybase, _L)]
        cb_x1 = x1[pl.ds(mybase + _L, _L)]
        cb_y1 = y1[pl.ds(mybase + _L, _L)]
        cb_x2 = x2[pl.ds(mybase + _L, _L)]
        cb_y2 = y2[pl.ds(mybase + _L, _L)]
        ca_ar = (ca_x2 - ca_x1) * (ca_y2 - ca_y1)
        cb_ar = (cb_x2 - cb_x1) * (cb_y2 - cb_y1)

        def scan_kept(t, sup):
            sa, sb = sup
            kx1v = _splat_gather(kx1, t)
            ky1v = _splat_gather(ky1, t)
            kx2v = _splat_gather(kx2, t)
            ky2v = _splat_gather(ky2, t)
            karv = _splat_gather(kar, t)

            wa = jnp.maximum(jnp.minimum(ca_x2, kx2v) - jnp.maximum(ca_x1, kx1v), 0.0)
            ha = jnp.maximum(jnp.minimum(ca_y2, ky2v) - jnp.maximum(ca_y1, ky1v), 0.0)
            ia = wa * ha
            sa = sa | (ia > 0.5 * (ca_ar + karv - ia))

            wb = jnp.maximum(jnp.minimum(cb_x2, kx2v) - jnp.maximum(cb_x1, kx1v), 0.0)
            hb = jnp.maximum(jnp.minimum(cb_y2, ky2v) - jnp.maximum(cb_y1, ky1v), 0.0)
            ib = wb * hb
            sb = sb | (ib > 0.5 * (cb_ar + karv - ib))
            return sa, sb

        sup_a, sup_b = lax.fori_loop(0, k_count, scan_kept, (ffalse, ffalse))
        stage[pl.ds(0, _L)] = jnp.where(sup_a, 0, 1).astype(jnp.int32)
        stage[pl.ds(_L, _L)] = jnp.where(sup_b, 0, 1).astype(jnp.int32)
        pltpu.sync_copy(stage, alive_sh.at[pl.ds(w * (_L * _U), _L * _U)])
        plsc.subcore_barrier()

        @pl.when(w == 0)
        def _():
            def initb(i, c):
                sl = pl.ds(i * _L, _L)
                far = jnp.full((_L,), _FAR, jnp.float32)
                bx1[sl] = far
                by1[sl] = far
                bx2[sl] = far
                by2[sl] = far
                bar_[sl] = fone
                bidx_l[sl] = jnp.zeros((_L,), jnp.int32)
                return c
            lax.fori_loop(0, _B // _L, initb, 0)
            pltpu.sync_copy(alive_sh, alive_l)

            def vreg_body(v, bk):
                av = alive_l[pl.ds(v * _L, _L)] != 0
                base_v = jnp.full((_L,), base + v * _L, jnp.int32)

                def process(bk_):
                    def wbody(carry):
                        bk2, m = carry
                        iv = plsc.all_reduce_ffs(m)
                        gv = base_v + iv
                        lane_iv = gv - base_v
                        m2 = m & (lanes != lane_iv)

                        cx1 = plsc.load_gather(x1, [gv])
                        cy1 = plsc.load_gather(y1, [gv])
                        cx2 = plsc.load_gather(x2, [gv])
                        cy2 = plsc.load_gather(y2, [gv])
                        car = (cx2 - cx1) * (cy2 - cy1)

                        def inner(t, sup):
                            sl = pl.ds(t * _L, _L)
                            wv = jnp.maximum(jnp.minimum(cx2, bx2[sl])
                                             - jnp.maximum(cx1, bx1[sl]), 0.0)
                            hv = jnp.maximum(jnp.minimum(cy2, by2[sl])
                                             - jnp.maximum(cy1, by1[sl]), 0.0)
                            ivr = wv * hv
                            return sup | (ivr > 0.5 * (car + bar_[sl] - ivr))

                        nvec = (bk2 + _L - 1) // _L
                        sup = lax.fori_loop(0, nvec, inner, ffalse)
                        suppressed = jnp.any(sup)

                        def append():
                            bv = jnp.full((_L,), bk2, jnp.int32)
                            plsc.store_scatter(bx1, [bv], cx1, mask=lane0)
                            plsc.store_scatter(by1, [bv], cy1, mask=lane0)
                            plsc.store_scatter(bx2, [bv], cx2, mask=lane0)
                            plsc.store_scatter(by2, [bv], cy2, mask=lane0)
                            plsc.store_scatter(bar_, [bv], car, mask=lane0)
                            plsc.store_scatter(bidx_l, [bv], gv, mask=lane0)
                            plsc.store_scatter(keepv, [gv], fone, mask=lane0)
                            return bk2 + 1

                        bk3 = lax.cond(suppressed, lambda: bk2, append)
                        return (bk3, m2)

                    return lax.while_loop(lambda c: jnp.any(c[1]), wbody,
                                          (bk_, av))[0]

                return lax.cond(jnp.any(av), process, lambda b: b, bk)

            bk = lax.fori_loop(0, _B // _L, vreg_body, 0)
            pltpu.sync_copy(bidx_l, bidx_sh)
            stage[pl.ds(0, _L)] = jnp.full((_L,), bk, jnp.int32)
            pltpu.sync_copy(stage.at[pl.ds(0, _L)], knew_sh)

        plsc.subcore_barrier()

        pltpu.sync_copy(knew_sh, knew_l)
        nb = jnp.max(knew_l[pl.ds(0, _L)])
        pltpu.sync_copy(bidx_sh, bidx_l)

        def append_delta(t, c):
            idxv = bidx_l[pl.ds(t * _L, _L)]
            m = (lanes + t * _L) < nb
            kvec = lanes + (k_count + t * _L)
            gx1 = plsc.load_gather(x1, [idxv])
            gy1 = plsc.load_gather(y1, [idxv])
            gx2 = plsc.load_gather(x2, [idxv])
            gy2 = plsc.load_gather(y2, [idxv])
            plsc.store_scatter(kx1, [kvec], gx1, mask=m)
            plsc.store_scatter(ky1, [kvec], gy1, mask=m)
            plsc.store_scatter(kx2, [kvec], gx2, mask=m)
            plsc.store_scatter(ky2, [kvec], gy2, mask=m)
            plsc.store_scatter(kar, [kvec], (gx2 - gx1) * (gy2 - gy1), mask=m)
            return c

        lax.fori_loop(0, (nb + _L - 1) // _L, append_delta, 0)
        return k_count + nb

    lax.fori_loop(0, _NB, block_body, 0)

    @pl.when(w == 0)
    def _():
        pltpu.sync_copy(keepv, keep_h)


@jax.jit
def _nms_keep01(x1s, y1s, x2s, y2s):
    mesh = plsc.VectorSubcoreMesh(core_axis_name="c", subcore_axis_name="s",
                                  num_cores=1)
    f = pl.kernel(
        _nms_body,
        out_type=jax.ShapeDtypeStruct((_NPAD,), jnp.float32),
        mesh=mesh,
        scratch_types=(
            [pltpu.VMEM((_NPAD,), jnp.float32) for _ in range(4)]
            + [pltpu.VMEM((_NPAD,), jnp.float32) for _ in range(5)]
            + [pltpu.VMEM((_B,), jnp.float32) for _ in range(5)]
            + [pltpu.VMEM((_B,), jnp.int32)]
            + [pltpu.VMEM((_B,), jnp.int32)]
            + [pltpu.VMEM((_NPAD,), jnp.float32)]
            + [pltpu.VMEM((_L * _U,), jnp.int32)]
            + [pltpu.VMEM((_L,), jnp.int32)]
            + [pltpu.VMEM_SHARED((_B,), jnp.int32)]
            + [pltpu.VMEM_SHARED((_B,), jnp.int32)]
            + [pltpu.VMEM_SHARED((_L,), jnp.int32)]
        ),
        compiler_params=pltpu.CompilerParams(needs_layout_passes=False),
    )
    return f(x1s, y1s, x2s, y2s)


def kernel(boxes, scores):
    order = jnp.argsort(-scores)
    boxes_sorted = boxes[order]
    scores_sorted = scores[order]
    pad = _NPAD - boxes_sorted.shape[0]
    bp = jnp.concatenate(
        [boxes_sorted, jnp.broadcast_to(boxes_sorted[0], (pad, 4))], axis=0)
    keep01 = _nms_keep01(bp[:, 0], bp[:, 1], bp[:, 2], bp[:, 3])[:_N]
    keep = keep01 > 0.5
    kept_scores = scores_sorted * keep01
    return kept_scores, keep, order


# speedup vs baseline: 36.8800x; 1.2422x over previous
import jax
import jax.numpy as jnp
from jax import lax
from jax.experimental import pallas as pl
from jax.experimental.pallas import tpu as pltpu
from jax.experimental.pallas import tpu_sc as plsc

_N = 5000
_NPAD = 5120
_L = 16
_NW = 16
_U = 2
_B = _NW * _L * _U
_NB = _NPAD // _B
_FAR = 2e9


def _splat_gather(ref, idx_scalar):
    iv = jnp.full((_L,), idx_scalar, jnp.int32)
    return plsc.load_gather(ref, [iv])


def _nms_body(x1h, y1h, x2h, y2h, keep_h,
              x1, y1, x2, y2,
              kx1, ky1, kx2, ky2, kar,
              stat_l, bidx_l, alive_l, keepv,
              stage, knew_l,
              alive_sh, stat_sh, bidx_sh, knew_sh):
    w = lax.axis_index("s")
    lanes = lax.broadcasted_iota(jnp.int32, (_L,), 0)
    lane0 = lanes == 0
    ffalse = lanes < 0
    fone = jnp.full((_L,), 1.0, jnp.float32)

    pltpu.sync_copy(x1h, x1)
    pltpu.sync_copy(y1h, y1)
    pltpu.sync_copy(x2h, x2)
    pltpu.sync_copy(y2h, y2)

    @pl.when(w == 0)
    def _():
        def initk(i, c):
            keepv[pl.ds(i * _L, _L)] = jnp.zeros((_L,), jnp.float32)
            return c
        lax.fori_loop(0, _NPAD // _L, initk, 0)

    def block_body(jb, k_count):
        base = jb * _B
        mybase = base + w * (_L * _U)

        ca_x1 = x1[pl.ds(mybase, _L)]
        ca_y1 = y1[pl.ds(mybase, _L)]
        ca_x2 = x2[pl.ds(mybase, _L)]
        ca_y2 = y2[pl.ds(mybase, _L)]
        cb_x1 = x1[pl.ds(mybase + _L, _L)]
        cb_y1 = y1[pl.ds(mybase + _L, _L)]
        cb_x2 = x2[pl.ds(mybase + _L, _L)]
        cb_y2 = y2[pl.ds(mybase + _L, _L)]
        ca_ar = (ca_x2 - ca_x1) * (ca_y2 - ca_y1)
        cb_ar = (cb_x2 - cb_x1) * (cb_y2 - cb_y1)

        def scan_kept(t, sup):
            sa, sb = sup
            kx1v = _splat_gather(kx1, t)
            ky1v = _splat_gather(ky1, t)
            kx2v = _splat_gather(kx2, t)
            ky2v = _splat_gather(ky2, t)
            karv = _splat_gather(kar, t)

            wa = jnp.maximum(jnp.minimum(ca_x2, kx2v) - jnp.maximum(ca_x1, kx1v), 0.0)
            ha = jnp.maximum(jnp.minimum(ca_y2, ky2v) - jnp.maximum(ca_y1, ky1v), 0.0)
            ia = wa * ha
            sa = sa | (ia > 0.5 * (ca_ar + karv - ia))

            wb = jnp.maximum(jnp.minimum(cb_x2, kx2v) - jnp.maximum(cb_x1, kx1v), 0.0)
            hb = jnp.maximum(jnp.minimum(cb_y2, ky2v) - jnp.maximum(cb_y1, ky1v), 0.0)
            ib = wb * hb
            sb = sb | (ib > 0.5 * (cb_ar + karv - ib))
            return sa, sb

        sup_a, sup_b = lax.fori_loop(0, k_count, scan_kept, (ffalse, ffalse))
        stage[pl.ds(0, _L)] = jnp.where(sup_a, 0, 1).astype(jnp.int32)
        stage[pl.ds(_L, _L)] = jnp.where(sup_b, 0, 1).astype(jnp.int32)
        pltpu.sync_copy(stage, alive_sh.at[pl.ds(w * (_L * _U), _L * _U)])
        plsc.subcore_barrier()

        pltpu.sync_copy(alive_sh, alive_l)
        mypos_a = lanes + w * (_L * _U)
        mypos_b = mypos_a + _L
        basev = jnp.full((_L,), base, jnp.int32)

        def a2_vreg(v, sup2):
            av = alive_l[pl.ds(v * _L, _L)] != 0

            def process(s2):
                def wbody(carry):
                    s2a, s2b, m = carry
                    iv = plsc.all_reduce_ffs(m)
                    qpos = jnp.full((_L,), v * _L, jnp.int32) + iv
                    m2 = m & (lanes != iv)
                    gq = basev + qpos
                    qx1 = plsc.load_gather(x1, [gq])
                    qy1 = plsc.load_gather(y1, [gq])
                    qx2 = plsc.load_gather(x2, [gq])
                    qy2 = plsc.load_gather(y2, [gq])
                    qar = (qx2 - qx1) * (qy2 - qy1)

                    wa = jnp.maximum(jnp.minimum(ca_x2, qx2) - jnp.maximum(ca_x1, qx1), 0.0)
                    ha = jnp.maximum(jnp.minimum(ca_y2, qy2) - jnp.maximum(ca_y1, qy1), 0.0)
                    ia = wa * ha
                    s2a = s2a | ((ia > 0.5 * (ca_ar + qar - ia)) & (qpos < mypos_a))

                    wb = jnp.maximum(jnp.minimum(cb_x2, qx2) - jnp.maximum(cb_x1, qx1), 0.0)
                    hb = jnp.maximum(jnp.minimum(cb_y2, qy2) - jnp.maximum(cb_y1, qy1), 0.0)
                    ib = wb * hb
                    s2b = s2b | ((ib > 0.5 * (cb_ar + qar - ib)) & (qpos < mypos_b))
                    return (s2a, s2b, m2)

                s2a, s2b, _ = lax.while_loop(lambda c: jnp.any(c[2]), wbody,
                                             (s2[0], s2[1], av))
                return (s2a, s2b)

            return lax.cond(jnp.any(av), process, lambda s: s, sup2)

        sup2_a, sup2_b = lax.fori_loop(0, _B // _L, a2_vreg, (ffalse, ffalse))
        stat_a = jnp.where(sup_a, 0, jnp.where(sup2_a, 2, 1)).astype(jnp.int32)
        stat_b = jnp.where(sup_b, 0, jnp.where(sup2_b, 2, 1)).astype(jnp.int32)
        stage[pl.ds(0, _L)] = stat_a
        stage[pl.ds(_L, _L)] = stat_b
        pltpu.sync_copy(stage, stat_sh.at[pl.ds(w * (_L * _U), _L * _U)])
        plsc.subcore_barrier()

        @pl.when(w == 0)
        def _():
            pltpu.sync_copy(stat_sh, stat_l)

            def initb(i, c):
                bidx_l[pl.ds(i * _L, _L)] = jnp.zeros((_L,), jnp.int32)
                return c
            lax.fori_loop(0, _B // _L, initb, 0)

            def res_vreg(v, c0):
                sv = stat_l[pl.ds(v * _L, _L)]
                um = sv == 2

                def process(c1):
                    def wbody(carry):
                        _, m = carry
                        iv = plsc.all_reduce_ffs(m)
                        m2 = m & (lanes != iv)
                        pv = jnp.full((_L,), v * _L, jnp.int32) + iv
                        gv = basev + pv
                        cx1 = plsc.load_gather(x1, [gv])
                        cy1 = plsc.load_gather(y1, [gv])
                        cx2 = plsc.load_gather(x2, [gv])
                        cy2 = plsc.load_gather(y2, [gv])
                        car = (cx2 - cx1) * (cy2 - cy1)

                        def inner(t, sup):
                            sl = pl.ds(base + t * _L, _L)
                            sv2 = stat_l[pl.ds(t * _L, _L)]
                            posv = lanes + t * _L
                            ex1 = x1[sl]
                            ey1 = y1[sl]
                            ex2 = x2[sl]
                            ey2 = y2[sl]
                            ear = (ex2 - ex1) * (ey2 - ey1)
                            wv = jnp.maximum(jnp.minimum(cx2, ex2)
                                             - jnp.maximum(cx1, ex1), 0.0)
                            hv = jnp.maximum(jnp.minimum(cy2, ey2)
                                             - jnp.maximum(cy1, ey1), 0.0)
                            ivr = wv * hv
                            hit = (ivr > 0.5 * (car + ear - ivr))
                            return sup | (hit & (sv2 == 1) & (posv < pv))

                        sup = lax.fori_loop(0, v + 1, inner, ffalse)
                        supi = jnp.any(sup).astype(jnp.int32)
                        val = jnp.full((_L,), 1, jnp.int32) - supi
                        plsc.store_scatter(stat_l, [pv], val, mask=lane0)
                        return (c1, m2)

                    return lax.while_loop(lambda c: jnp.any(c[1]), wbody,
                                          (c1, um))[0]

                return lax.cond(jnp.any(um), process, lambda c: c, c0)

            lax.fori_loop(0, _B // _L, res_vreg, 0)

            def cmp_vreg(v, nb0):
                sv = stat_l[pl.ds(v * _L, _L)]
                km = sv == 1

                def process(nb1):
                    def wbody(carry):
                        nbc, m = carry
                        iv = plsc.all_reduce_ffs(m)
                        m2 = m & (lanes != iv)
                        pv = jnp.full((_L,), v * _L, jnp.int32) + iv
                        nv = jnp.full((_L,), nbc, jnp.int32)
                        plsc.store_scatter(bidx_l, [nv], pv, mask=lane0)
                        plsc.store_scatter(keepv, [basev + pv], fone, mask=lane0)
                        return (nbc + 1, m2)

                    return lax.while_loop(lambda c: jnp.any(c[1]), wbody,
                                          (nb1, km))[0]

                return lax.cond(jnp.any(km), process, lambda n: n, nb0)

            bk = lax.fori_loop(0, _B // _L, cmp_vreg, 0)
            pltpu.sync_copy(bidx_l, bidx_sh)
            stage[pl.ds(0, _L)] = jnp.full((_L,), bk, jnp.int32)
            pltpu.sync_copy(stage.at[pl.ds(0, _L)], knew_sh)

        plsc.subcore_barrier()

        pltpu.sync_copy(knew_sh, knew_l)
        nb = jnp.max(knew_l[pl.ds(0, _L)])
        pltpu.sync_copy(bidx_sh, bidx_l)

        def append_delta(t, c):
            idxv = basev + bidx_l[pl.ds(t * _L, _L)]
            m = (lanes + t * _L) < nb
            kvec = lanes + (k_count + t * _L)
            gx1 = plsc.load_gather(x1, [idxv])
            gy1 = plsc.load_gather(y1, [idxv])
            gx2 = plsc.load_gather(x2, [idxv])
            gy2 = plsc.load_gather(y2, [idxv])
            plsc.store_scatter(kx1, [kvec], gx1, mask=m)
            plsc.store_scatter(ky1, [kvec], gy1, mask=m)
            plsc.store_scatter(kx2, [kvec], gx2, mask=m)
            plsc.store_scatter(ky2, [kvec], gy2, mask=m)
            plsc.store_scatter(kar, [kvec], (gx2 - gx1) * (gy2 - gy1), mask=m)
            return c

        lax.fori_loop(0, (nb + _L - 1) // _L, append_delta, 0)
        return k_count + nb

    lax.fori_loop(0, _NB, block_body, 0)

    @pl.when(w == 0)
    def _():
        pltpu.sync_copy(keepv, keep_h)


@jax.jit
def _nms_keep01(x1s, y1s, x2s, y2s):
    mesh = plsc.VectorSubcoreMesh(core_axis_name="c", subcore_axis_name="s",
                                  num_cores=1)
    f = pl.kernel(
        _nms_body,
        out_type=jax.ShapeDtypeStruct((_NPAD,), jnp.float32),
        mesh=mesh,
        scratch_types=(
            [pltpu.VMEM((_NPAD,), jnp.float32) for _ in range(4)]
            + [pltpu.VMEM((_NPAD,), jnp.float32) for _ in range(5)]
            + [pltpu.VMEM((_B,), jnp.int32)]
            + [pltpu.VMEM((_B,), jnp.int32)]
            + [pltpu.VMEM((_B,), jnp.int32)]
            + [pltpu.VMEM((_NPAD,), jnp.float32)]
            + [pltpu.VMEM((_L * _U,), jnp.int32)]
            + [pltpu.VMEM((_L,), jnp.int32)]
            + [pltpu.VMEM_SHARED((_B,), jnp.int32)]
            + [pltpu.VMEM_SHARED((_B,), jnp.int32)]
            + [pltpu.VMEM_SHARED((_B,), jnp.int32)]
            + [pltpu.VMEM_SHARED((_L,), jnp.int32)]
        ),
        compiler_params=pltpu.CompilerParams(needs_layout_passes=False),
    )
    return f(x1s, y1s, x2s, y2s)


def kernel(boxes, scores):
    order = jnp.argsort(-scores)
    boxes_sorted = boxes[order]
    scores_sorted = scores[order]
    pad = _NPAD - boxes_sorted.shape[0]
    bp = jnp.concatenate(
        [boxes_sorted, jnp.broadcast_to(boxes_sorted[0], (pad, 4))], axis=0)
    keep01 = _nms_keep01(bp[:, 0], bp[:, 1], bp[:, 2], bp[:, 3])[:_N]
    keep = keep01 > 0.5
    kept_scores = scores_sorted * keep01
    return kept_scores, keep, order
